# baseline (speedup 1.0000x reference)
import functools

import jax
import jax.numpy as jnp
from jax import lax
from jax.experimental import pallas as pl
from jax.experimental.pallas import tpu as pltpu
from jax.experimental.pallas import tpu_sc as plsc

N_NODES = 10000
N_EDGES = 160000
D_FEAT = 256
D_HALF = 128
D_EDGE = 16
D_HID = 1024

E_BLK = 128
N_SUBCORES = 16
N_CORES = 2
N_WORKERS = N_CORES * N_SUBCORES

N_CHUNKS = 2
E_CHUNK = N_EDGES // N_CHUNKS
CBLKS = E_CHUNK // E_BLK


def _mish(x):
    u = 1.0 + jnp.exp(jnp.minimum(x, 20.0))
    uu = u * u
    return x * ((uu - 1.0) / (uu + 1.0))


def _sc_gather(nodes_packed, senders2d, ci):
    mesh = plsc.VectorSubcoreMesh(core_axis_name="c", subcore_axis_name="s")
    off = ci * CBLKS

    @functools.partial(
        pl.kernel,
        out_type=jax.ShapeDtypeStruct((E_CHUNK, D_HALF), jnp.uint32),
        mesh=mesh,
    )
    def k(nodes_hbm, idx_hbm, out_hbm):
        def body(i_vmem, o_vmem):
            pltpu.sync_copy(nodes_hbm.at[i_vmem.at[0]], o_vmem)

        pltpu.emit_pipeline(
            body,
            grid=(CBLKS,),
            in_specs=[pl.BlockSpec((1, E_BLK), lambda i: (0, i + off))],
            out_specs=[pl.BlockSpec((E_BLK, D_HALF), lambda i: (i, 0))],
            core_axis_name=("c", "s"),
            dimension_semantics=(pltpu.PARALLEL,),
        )(idx_hbm, out_hbm)

    return k(nodes_packed, senders2d)


def _tc_messages(sent, edges, W_e, b_e2d, ci):
    BLK = 1000
    grid = (E_CHUNK // BLK,)
    roff = ci * (E_CHUNK // BLK)

    def body(sent_ref, edges_ref, we_ref, be_ref, lo_ref, hi_ref):
        emb = jnp.dot(edges_ref[...], we_ref[...],
                      preferred_element_type=jnp.float32)
        bits = sent_ref[...]
        sent_lo = lax.bitcast_convert_type(bits << 16, jnp.float32)
        sent_hi = lax.bitcast_convert_type(bits & jnp.uint32(0xFFFF0000),
                                           jnp.float32)
        eb = emb + be_ref[...]
        lo_ref[...] = _mish(sent_lo + eb[:, :D_HALF])
        hi_ref[...] = _mish(sent_hi + eb[:, D_HALF:])

    return pl.pallas_call(
        body,
        grid=grid,
        in_specs=[
            pl.BlockSpec((BLK, D_HALF), lambda i: (i, 0)),
            pl.BlockSpec((BLK, D_EDGE), lambda i: (i + roff, 0)),
            pl.BlockSpec((D_EDGE, D_FEAT), lambda i: (0, 0)),
            pl.BlockSpec((1, D_FEAT), lambda i: (0, 0)),
        ],
        out_specs=[
            pl.BlockSpec((BLK, D_HALF), lambda i: (i, 0)),
            pl.BlockSpec((BLK, D_HALF), lambda i: (i, 0)),
        ],
        out_shape=[
            jax.ShapeDtypeStruct((E_CHUNK, D_HALF), jnp.float32),
            jax.ShapeDtypeStruct((E_CHUNK, D_HALF), jnp.float32),
        ],
    )(sent, edges, W_e, b_e2d)


def _sc_scatter(mlo, mhi, receivers2d, ci):
    ROW_BLK = 80
    n_row_blks = N_NODES // ROW_BLK
    mesh = plsc.VectorSubcoreMesh(core_axis_name="c", subcore_axis_name="s")
    off = ci * CBLKS

    @functools.partial(
        pl.kernel,
        out_type=(
            jax.ShapeDtypeStruct((N_NODES, D_HALF), jnp.float32),
            jax.ShapeDtypeStruct((N_NODES, D_HALF), jnp.float32),
        ),
        mesh=mesh,
        scratch_types=[
            pltpu.VMEM((ROW_BLK, D_HALF), jnp.float32),
            pltpu.VMEM_SHARED((N_NODES, D_HALF), jnp.float32),
        ],
    )
    def k(mlo_hbm, mhi_hbm, recv_hbm, olo_hbm, ohi_hbm, zbuf, acc):
        cid = lax.axis_index("c")
        sid = lax.axis_index("s")

        @pl.loop(0, ROW_BLK)
        def _(r):
            @pl.loop(0, D_HALF, step=16)
            def _(cc):
                zbuf.at[pl.ds(r, 1), pl.ds(cc, 16)][...] = (
                    jnp.zeros((1, 16), jnp.float32))

        @pl.loop(sid, n_row_blks, step=N_SUBCORES)
        def _(t):
            pltpu.sync_copy(zbuf, acc.at[pl.ds(t * ROW_BLK, ROW_BLK)])

        plsc.subcore_barrier()

        def halfwork(m_hbm, o_hbm):
            def body(i_vmem, m_vmem):
                pltpu.sync_copy(m_vmem, acc.at[i_vmem.at[0]], add=True)

            pltpu.emit_pipeline(
                body,
                grid=(CBLKS,),
                in_specs=[
                    pl.BlockSpec((1, E_BLK), lambda i: (0, i + off)),
                    pl.BlockSpec((E_BLK, D_HALF), lambda i: (i, 0)),
                ],
                out_specs=[],
                core_axis_name="s",
                dimension_semantics=(pltpu.PARALLEL,),
            )(recv_hbm, m_hbm)

            plsc.subcore_barrier()

            @pl.loop(sid, n_row_blks, step=N_SUBCORES)
            def _(t):
                pltpu.sync_copy(acc.at[pl.ds(t * ROW_BLK, ROW_BLK)],
                                o_hbm.at[pl.ds(t * ROW_BLK, ROW_BLK)])

        @pl.when(cid == 0)
        def _():
            halfwork(mlo_hbm, olo_hbm)

        @pl.when(cid == 1)
        def _():
            halfwork(mhi_hbm, ohi_hbm)

    return k(mlo, mhi, receivers2d)


def _tc_mlp(nodes, r_parts, epsilon, W1, b1, W2, b2):
    BLK = 1000
    grid = (N_NODES // BLK,)

    def body(*refs):
        nodes_ref = refs[0]
        part_refs = refs[1:1 + 2 * N_CHUNKS]
        eps_ref, w1_ref, b1_ref, w2_ref, b2_ref, o_ref = refs[1 + 2 * N_CHUNKS:]
        rlo = part_refs[0][...]
        rhi = part_refs[1][...]
        for ci in range(1, N_CHUNKS):
            rlo = rlo + part_refs[2 * ci][...]
            rhi = rhi + part_refs[2 * ci + 1][...]
        received = jnp.concatenate([rlo, rhi], axis=1)
        h = (1.0 + eps_ref[...]) * nodes_ref[...] + received
        a = _mish(jnp.dot(h.astype(jnp.bfloat16),
                          w1_ref[...].astype(jnp.bfloat16),
                          preferred_element_type=jnp.float32) + b1_ref[...])
        o_ref[...] = jnp.dot(a.astype(jnp.bfloat16),
                             w2_ref[...].astype(jnp.bfloat16),
                             preferred_element_type=jnp.float32) + b2_ref[...]

    return pl.pallas_call(
        body,
        grid=grid,
        in_specs=[pl.BlockSpec((BLK, D_FEAT), lambda i: (i, 0))]
        + [pl.BlockSpec((BLK, D_HALF), lambda i: (i, 0))] * (2 * N_CHUNKS)
        + [
            pl.BlockSpec((1, 1), lambda i: (0, 0)),
            pl.BlockSpec((D_FEAT, D_HID), lambda i: (0, 0)),
            pl.BlockSpec((1, D_HID), lambda i: (0, 0)),
            pl.BlockSpec((D_HID, D_FEAT), lambda i: (0, 0)),
            pl.BlockSpec((1, D_FEAT), lambda i: (0, 0)),
        ],
        out_specs=pl.BlockSpec((BLK, D_FEAT), lambda i: (i, 0)),
        out_shape=jax.ShapeDtypeStruct((N_NODES, D_FEAT), jnp.float32),
    )(nodes, *r_parts, epsilon, W1, b1.reshape(1, D_HID), W2,
      b2.reshape(1, D_FEAT))


def kernel(nodes, edges, senders, receivers, W_e, b_e, epsilon, W1, b1, W2, b2):
    senders2d = senders.reshape(1, N_EDGES)
    b_e2d = b_e.reshape(1, D_FEAT)
    lo_bits = lax.bitcast_convert_type(
        nodes[:, :D_HALF].astype(jnp.bfloat16), jnp.uint16).astype(jnp.uint32)
    hi_bits = lax.bitcast_convert_type(
        nodes[:, D_HALF:].astype(jnp.bfloat16), jnp.uint16).astype(jnp.uint32)
    nodes_packed = lo_bits | (hi_bits << 16)
    recv2d = receivers.reshape(1, N_EDGES)
    r_parts = []
    for ci in range(N_CHUNKS):
        sent = _sc_gather(nodes_packed, senders2d, ci)
        mlo, mhi = _tc_messages(sent, edges, W_e, b_e2d, ci)
        r_parts.extend(_sc_scatter(mlo, mhi, recv2d, ci))
    return _tc_mlp(nodes, r_parts, epsilon, W1, b1, W2, b2)


# speedup vs baseline: 1.1155x; 1.1155x over previous
import functools

import jax
import jax.numpy as jnp
from jax import lax
from jax.experimental import pallas as pl
from jax.experimental.pallas import tpu as pltpu
from jax.experimental.pallas import tpu_sc as plsc

N_NODES = 10000
N_EDGES = 160000
D_FEAT = 256
D_HALF = 128
D_EDGE = 16
D_HID = 1024

E_BLK = 128
N_SUBCORES = 16
N_CORES = 2
N_WORKERS = N_CORES * N_SUBCORES

N_CHUNKS = 2
E_CHUNK = N_EDGES // N_CHUNKS
CBLKS = E_CHUNK // E_BLK


def _mish(x):
    u = 1.0 + jnp.exp(jnp.minimum(x, 20.0))
    return x - (x + x) / (u * u + 1.0)


def _sc_gather(nodes_packed, senders2d, ci):
    mesh = plsc.VectorSubcoreMesh(core_axis_name="c", subcore_axis_name="s")
    off = ci * CBLKS

    @functools.partial(
        pl.kernel,
        out_type=jax.ShapeDtypeStruct((E_CHUNK, D_HALF), jnp.uint32),
        mesh=mesh,
    )
    def k(nodes_hbm, idx_hbm, out_hbm):
        def body(i_vmem, o_vmem):
            pltpu.sync_copy(nodes_hbm.at[i_vmem.at[0]], o_vmem)

        pltpu.emit_pipeline(
            body,
            grid=(CBLKS,),
            in_specs=[pl.BlockSpec((1, E_BLK), lambda i: (0, i + off))],
            out_specs=[pl.BlockSpec((E_BLK, D_HALF), lambda i: (i, 0))],
            core_axis_name=("c", "s"),
            dimension_semantics=(pltpu.PARALLEL,),
        )(idx_hbm, out_hbm)

    return k(nodes_packed, senders2d)


def _tc_messages(sent, edges, W_e, b_e2d, ci):
    BLK = 2000
    grid = (E_CHUNK // BLK,)
    roff = ci * (E_CHUNK // BLK)

    def body(sent_ref, edges_ref, we_ref, be_ref, lo_ref, hi_ref):
        emb = jnp.dot(edges_ref[...], we_ref[...],
                      preferred_element_type=jnp.float32)
        bits = sent_ref[...]
        sent_lo = lax.bitcast_convert_type(bits << 16, jnp.float32)
        sent_hi = lax.bitcast_convert_type(bits & jnp.uint32(0xFFFF0000),
                                           jnp.float32)
        eb = emb + be_ref[...]
        lo_ref[...] = _mish(sent_lo + eb[:, :D_HALF])
        hi_ref[...] = _mish(sent_hi + eb[:, D_HALF:])

    return pl.pallas_call(
        body,
        grid=grid,
        in_specs=[
            pl.BlockSpec((BLK, D_HALF), lambda i: (i, 0)),
            pl.BlockSpec((BLK, D_EDGE), lambda i: (i + roff, 0)),
            pl.BlockSpec((D_EDGE, D_FEAT), lambda i: (0, 0)),
            pl.BlockSpec((1, D_FEAT), lambda i: (0, 0)),
        ],
        out_specs=[
            pl.BlockSpec((BLK, D_HALF), lambda i: (i, 0)),
            pl.BlockSpec((BLK, D_HALF), lambda i: (i, 0)),
        ],
        out_shape=[
            jax.ShapeDtypeStruct((E_CHUNK, D_HALF), jnp.float32),
            jax.ShapeDtypeStruct((E_CHUNK, D_HALF), jnp.float32),
        ],
    )(sent, edges, W_e, b_e2d)


def _sc_scatter(mlo, mhi, receivers2d, ci):
    ROW_BLK = 80
    n_row_blks = N_NODES // ROW_BLK
    mesh = plsc.VectorSubcoreMesh(core_axis_name="c", subcore_axis_name="s")
    off = ci * CBLKS

    @functools.partial(
        pl.kernel,
        out_type=(
            jax.ShapeDtypeStruct((N_NODES, D_HALF), jnp.float32),
            jax.ShapeDtypeStruct((N_NODES, D_HALF), jnp.float32),
        ),
        mesh=mesh,
        scratch_types=[
            pltpu.VMEM((ROW_BLK, D_HALF), jnp.float32),
            pltpu.VMEM_SHARED((N_NODES, D_HALF), jnp.float32),
        ],
    )
    def k(mlo_hbm, mhi_hbm, recv_hbm, olo_hbm, ohi_hbm, zbuf, acc):
        cid = lax.axis_index("c")
        sid = lax.axis_index("s")

        @pl.loop(0, ROW_BLK)
        def _(r):
            @pl.loop(0, D_HALF, step=16)
            def _(cc):
                zbuf.at[pl.ds(r, 1), pl.ds(cc, 16)][...] = (
                    jnp.zeros((1, 16), jnp.float32))

        @pl.loop(sid, n_row_blks, step=N_SUBCORES)
        def _(t):
            pltpu.sync_copy(zbuf, acc.at[pl.ds(t * ROW_BLK, ROW_BLK)])

        plsc.subcore_barrier()

        def halfwork(m_hbm, o_hbm):
            def body(i_vmem, m_vmem):
                pltpu.sync_copy(m_vmem, acc.at[i_vmem.at[0]], add=True)

            pltpu.emit_pipeline(
                body,
                grid=(CBLKS,),
                in_specs=[
                    pl.BlockSpec((1, E_BLK), lambda i: (0, i + off)),
                    pl.BlockSpec((E_BLK, D_HALF), lambda i: (i, 0)),
                ],
                out_specs=[],
                core_axis_name="s",
                dimension_semantics=(pltpu.PARALLEL,),
            )(recv_hbm, m_hbm)

            plsc.subcore_barrier()

            @pl.loop(sid, n_row_blks, step=N_SUBCORES)
            def _(t):
                pltpu.sync_copy(acc.at[pl.ds(t * ROW_BLK, ROW_BLK)],
                                o_hbm.at[pl.ds(t * ROW_BLK, ROW_BLK)])

        @pl.when(cid == 0)
        def _():
            halfwork(mlo_hbm, olo_hbm)

        @pl.when(cid == 1)
        def _():
            halfwork(mhi_hbm, ohi_hbm)

    return k(mlo, mhi, receivers2d)


def _tc_mlp(nodes, r_parts, epsilon, W1, b1, W2, b2):
    BLK = 1000
    grid = (N_NODES // BLK,)

    def body(*refs):
        nodes_ref = refs[0]
        part_refs = refs[1:1 + 2 * N_CHUNKS]
        eps_ref, w1_ref, b1_ref, w2_ref, b2_ref, o_ref = refs[1 + 2 * N_CHUNKS:]
        rlo = part_refs[0][...]
        rhi = part_refs[1][...]
        for ci in range(1, N_CHUNKS):
            rlo = rlo + part_refs[2 * ci][...]
            rhi = rhi + part_refs[2 * ci + 1][...]
        received = jnp.concatenate([rlo, rhi], axis=1)
        h = (1.0 + eps_ref[...]) * nodes_ref[...] + received
        a = _mish(jnp.dot(h, w1_ref[...],
                          preferred_element_type=jnp.float32) + b1_ref[...])
        o_ref[...] = jnp.dot(a, w2_ref[...],
                             preferred_element_type=jnp.float32) + b2_ref[...]

    return pl.pallas_call(
        body,
        grid=grid,
        in_specs=[pl.BlockSpec((BLK, D_FEAT), lambda i: (i, 0))]
        + [pl.BlockSpec((BLK, D_HALF), lambda i: (i, 0))] * (2 * N_CHUNKS)
        + [
            pl.BlockSpec((1, 1), lambda i: (0, 0)),
            pl.BlockSpec((D_FEAT, D_HID), lambda i: (0, 0)),
            pl.BlockSpec((1, D_HID), lambda i: (0, 0)),
            pl.BlockSpec((D_HID, D_FEAT), lambda i: (0, 0)),
            pl.BlockSpec((1, D_FEAT), lambda i: (0, 0)),
        ],
        out_specs=pl.BlockSpec((BLK, D_FEAT), lambda i: (i, 0)),
        out_shape=jax.ShapeDtypeStruct((N_NODES, D_FEAT), jnp.float32),
    )(nodes, *r_parts, epsilon, W1, b1.reshape(1, D_HID), W2,
      b2.reshape(1, D_FEAT))


def kernel(nodes, edges, senders, receivers, W_e, b_e, epsilon, W1, b1, W2, b2):
    senders2d = senders.reshape(1, N_EDGES)
    b_e2d = b_e.reshape(1, D_FEAT)
    lo_bits = lax.bitcast_convert_type(
        nodes[:, :D_HALF].astype(jnp.bfloat16), jnp.uint16).astype(jnp.uint32)
    hi_bits = lax.bitcast_convert_type(
        nodes[:, D_HALF:].astype(jnp.bfloat16), jnp.uint16).astype(jnp.uint32)
    nodes_packed = lo_bits | (hi_bits << 16)
    recv2d = receivers.reshape(1, N_EDGES)
    r_parts = []
    for ci in range(N_CHUNKS):
        sent = _sc_gather(nodes_packed, senders2d, ci)
        mlo, mhi = _tc_messages(sent, edges, W_e, b_e2d, ci)
        r_parts.extend(_sc_scatter(mlo, mhi, recv2d, ci))
    return _tc_mlp(nodes, r_parts, epsilon, W1, b1, W2, b2)


# speedup vs baseline: 1.1176x; 1.0018x over previous
import functools

import jax
import jax.numpy as jnp
from jax import lax
from jax.experimental import pallas as pl
from jax.experimental.pallas import tpu as pltpu
from jax.experimental.pallas import tpu_sc as plsc

N_NODES = 10000
N_EDGES = 160000
D_FEAT = 256
D_HALF = 128
D_EDGE = 16
D_HID = 1024

E_BLK = 128
N_SUBCORES = 16
N_CORES = 2
N_WORKERS = N_CORES * N_SUBCORES

N_CHUNKS = 2
E_CHUNK = N_EDGES // N_CHUNKS
CBLKS = E_CHUNK // E_BLK


def _mish(x):
    u = 1.0 + jnp.exp(jnp.minimum(x, 20.0))
    return x - (x + x) / (u * u + 1.0)


def _sc_gather(nodes_packed, senders2d, ci):
    mesh = plsc.VectorSubcoreMesh(core_axis_name="c", subcore_axis_name="s")
    off = ci * CBLKS

    @functools.partial(
        pl.kernel,
        out_type=jax.ShapeDtypeStruct((E_CHUNK, D_HALF), jnp.uint32),
        mesh=mesh,
    )
    def k(nodes_hbm, idx_hbm, out_hbm):
        def body(i_vmem, o_vmem):
            pltpu.sync_copy(nodes_hbm.at[i_vmem.at[0]], o_vmem)

        pltpu.emit_pipeline(
            body,
            grid=(CBLKS,),
            in_specs=[pl.BlockSpec((1, E_BLK), lambda i: (0, i + off))],
            out_specs=[pl.BlockSpec((E_BLK, D_HALF), lambda i: (i, 0))],
            core_axis_name=("c", "s"),
            dimension_semantics=(pltpu.PARALLEL,),
        )(idx_hbm, out_hbm)

    return k(nodes_packed, senders2d)


def _tc_messages(sent, edges, W_e, b_e2d, ci):
    BLK = 2000
    grid = (E_CHUNK // BLK,)
    roff = ci * (E_CHUNK // BLK)

    def body(sent_ref, edges_ref, we_ref, be_ref, lo_ref, hi_ref):
        emb = jnp.dot(edges_ref[...], we_ref[...],
                      preferred_element_type=jnp.float32)
        bits = sent_ref[...]
        sent_lo = lax.bitcast_convert_type(bits << 16, jnp.float32)
        sent_hi = lax.bitcast_convert_type(bits & jnp.uint32(0xFFFF0000),
                                           jnp.float32)
        eb = emb + be_ref[...]
        lo_ref[...] = _mish(sent_lo + eb[:, :D_HALF])
        hi_ref[...] = _mish(sent_hi + eb[:, D_HALF:])

    return pl.pallas_call(
        body,
        grid=grid,
        in_specs=[
            pl.BlockSpec((BLK, D_HALF), lambda i: (i, 0)),
            pl.BlockSpec((BLK, D_EDGE), lambda i: (i + roff, 0)),
            pl.BlockSpec((D_EDGE, D_FEAT), lambda i: (0, 0)),
            pl.BlockSpec((1, D_FEAT), lambda i: (0, 0)),
        ],
        out_specs=[
            pl.BlockSpec((BLK, D_HALF), lambda i: (i, 0)),
            pl.BlockSpec((BLK, D_HALF), lambda i: (i, 0)),
        ],
        out_shape=[
            jax.ShapeDtypeStruct((E_CHUNK, D_HALF), jnp.float32),
            jax.ShapeDtypeStruct((E_CHUNK, D_HALF), jnp.float32),
        ],
    )(sent, edges, W_e, b_e2d)


def _sc_scatter(mlo, mhi, receivers2d, ci):
    ROW_BLK = 80
    n_row_blks = N_NODES // ROW_BLK
    mesh = plsc.VectorSubcoreMesh(core_axis_name="c", subcore_axis_name="s")
    off = ci * CBLKS

    @functools.partial(
        pl.kernel,
        out_type=(
            jax.ShapeDtypeStruct((N_NODES, D_HALF), jnp.float32),
            jax.ShapeDtypeStruct((N_NODES, D_HALF), jnp.float32),
        ),
        mesh=mesh,
        scratch_types=[
            pltpu.VMEM((ROW_BLK, D_HALF), jnp.float32),
            pltpu.VMEM_SHARED((N_NODES, D_HALF), jnp.float32),
        ],
    )
    def k(mlo_hbm, mhi_hbm, recv_hbm, olo_hbm, ohi_hbm, zbuf, acc):
        cid = lax.axis_index("c")
        sid = lax.axis_index("s")

        @pl.loop(0, ROW_BLK)
        def _(r):
            @pl.loop(0, D_HALF, step=16)
            def _(cc):
                zbuf.at[pl.ds(r, 1), pl.ds(cc, 16)][...] = (
                    jnp.zeros((1, 16), jnp.float32))

        @pl.loop(sid, n_row_blks, step=N_SUBCORES)
        def _(t):
            pltpu.sync_copy(zbuf, acc.at[pl.ds(t * ROW_BLK, ROW_BLK)])

        plsc.subcore_barrier()

        def halfwork(m_hbm, o_hbm):
            def body(i_vmem, m_vmem):
                pltpu.sync_copy(m_vmem, acc.at[i_vmem.at[0]], add=True)

            pltpu.emit_pipeline(
                body,
                grid=(CBLKS,),
                in_specs=[
                    pl.BlockSpec((1, E_BLK), lambda i: (0, i + off)),
                    pl.BlockSpec((E_BLK, D_HALF), lambda i: (i, 0)),
                ],
                out_specs=[],
                core_axis_name="s",
                dimension_semantics=(pltpu.PARALLEL,),
            )(recv_hbm, m_hbm)

            plsc.subcore_barrier()

            @pl.loop(sid, n_row_blks, step=N_SUBCORES)
            def _(t):
                pltpu.sync_copy(acc.at[pl.ds(t * ROW_BLK, ROW_BLK)],
                                o_hbm.at[pl.ds(t * ROW_BLK, ROW_BLK)])

        @pl.when(cid == 0)
        def _():
            halfwork(mlo_hbm, olo_hbm)

        @pl.when(cid == 1)
        def _():
            halfwork(mhi_hbm, ohi_hbm)

    return k(mlo, mhi, receivers2d)


def _tc_mlp(nodes, r_parts, epsilon, W1, b1, W2, b2):
    BLK = 2000
    grid = (N_NODES // BLK,)

    def body(*refs):
        nodes_ref = refs[0]
        part_refs = refs[1:1 + 2 * N_CHUNKS]
        eps_ref, w1_ref, b1_ref, w2_ref, b2_ref, o_ref = refs[1 + 2 * N_CHUNKS:]
        rlo = part_refs[0][...]
        rhi = part_refs[1][...]
        for ci in range(1, N_CHUNKS):
            rlo = rlo + part_refs[2 * ci][...]
            rhi = rhi + part_refs[2 * ci + 1][...]
        received = jnp.concatenate([rlo, rhi], axis=1)
        h = (1.0 + eps_ref[...]) * nodes_ref[...] + received
        a = _mish(jnp.dot(h, w1_ref[...],
                          preferred_element_type=jnp.float32) + b1_ref[...])
        o_ref[...] = jnp.dot(a, w2_ref[...],
                             preferred_element_type=jnp.float32) + b2_ref[...]

    return pl.pallas_call(
        body,
        grid=grid,
        in_specs=[pl.BlockSpec((BLK, D_FEAT), lambda i: (i, 0))]
        + [pl.BlockSpec((BLK, D_HALF), lambda i: (i, 0))] * (2 * N_CHUNKS)
        + [
            pl.BlockSpec((1, 1), lambda i: (0, 0)),
            pl.BlockSpec((D_FEAT, D_HID), lambda i: (0, 0)),
            pl.BlockSpec((1, D_HID), lambda i: (0, 0)),
            pl.BlockSpec((D_HID, D_FEAT), lambda i: (0, 0)),
            pl.BlockSpec((1, D_FEAT), lambda i: (0, 0)),
        ],
        out_specs=pl.BlockSpec((BLK, D_FEAT), lambda i: (i, 0)),
        out_shape=jax.ShapeDtypeStruct((N_NODES, D_FEAT), jnp.float32),
    )(nodes, *r_parts, epsilon, W1, b1.reshape(1, D_HID), W2,
      b2.reshape(1, D_FEAT))


def kernel(nodes, edges, senders, receivers, W_e, b_e, epsilon, W1, b1, W2, b2):
    senders2d = senders.reshape(1, N_EDGES)
    b_e2d = b_e.reshape(1, D_FEAT)
    lo_bits = lax.bitcast_convert_type(
        nodes[:, :D_HALF].astype(jnp.bfloat16), jnp.uint16).astype(jnp.uint32)
    hi_bits = lax.bitcast_convert_type(
        nodes[:, D_HALF:].astype(jnp.bfloat16), jnp.uint16).astype(jnp.uint32)
    nodes_packed = lo_bits | (hi_bits << 16)
    recv2d = receivers.reshape(1, N_EDGES)
    r_parts = []
    for ci in range(N_CHUNKS):
        sent = _sc_gather(nodes_packed, senders2d, ci)
        mlo, mhi = _tc_messages(sent, edges, W_e, b_e2d, ci)
        r_parts.extend(_sc_scatter(mlo, mhi, recv2d, ci))
    return _tc_mlp(nodes, r_parts, epsilon, W1, b1, W2, b2)
